# baseline (speedup 1.0000x reference)
import functools

import jax
import jax.numpy as jnp
from jax import lax
from jax.experimental import pallas as pl
from jax.experimental.pallas import tpu as pltpu
from jax.experimental.pallas import tpu_sc as plsc

B = 16384
EMB = 128
NC = 2
NS = 16
NW = NC * NS
PER_W = B // NW
CHUNK = 128
HALF = CHUNK // 2
NCHUNK = PER_W // CHUNK
LANES = 16
GROUPS = CHUNK // LANES
HGROUPS = GROUPS // 2


def _fm_body(uf, vf, ub, ib, uid, iid, gb,
             pred_out, ctr_out,
             uidall, iidall, urows, vrows, ubias, ibias, predall,
             ctrall, gbv,
             sem_ul0, sem_ul1, sem_uh0, sem_uh1,
             sem_vl0, sem_vl1, sem_vh0, sem_vh1,
             sem_ub0, sem_ub1, sem_ib0, sem_ib1,
             sem_wp, sem_wc, sem_p0, sem_p1, sem_p2, sem_p3, sem_p4):
    cid = lax.axis_index("c")
    sid = lax.axis_index("s")
    wid = sid * NC + cid
    base = wid * PER_W

    sem_ul = (sem_ul0, sem_ul1)
    sem_uh = (sem_uh0, sem_uh1)
    sem_vl = (sem_vl0, sem_vl1)
    sem_vh = (sem_vh0, sem_vh1)
    sem_ub = (sem_ub0, sem_ub1)
    sem_ib = (sem_ib0, sem_ib1)

    cuid0 = pltpu.async_copy(uid.at[pl.ds(base, CHUNK)],
                             uidall.at[pl.ds(0, CHUNK)], sem_p1)
    ciid0 = pltpu.async_copy(iid.at[pl.ds(base, CHUNK)],
                             iidall.at[pl.ds(0, CHUNK)], sem_p2)
    cg = pltpu.async_copy(gb, gbv, sem_p0)
    cuidr = pltpu.async_copy(uid.at[pl.ds(base + CHUNK, PER_W - CHUNK)],
                             uidall.at[pl.ds(CHUNK, PER_W - CHUNK)], sem_p3)
    ciidr = pltpu.async_copy(iid.at[pl.ds(base + CHUNK, PER_W - CHUNK)],
                             iidall.at[pl.ds(CHUNK, PER_W - CHUNK)], sem_p4)

    def issue(c):
        b = c % 2
        idxu_lo = uidall.at[pl.ds(c * CHUNK, HALF)]
        idxu_hi = uidall.at[pl.ds(c * CHUNK + HALF, HALF)]
        idxi_lo = iidall.at[pl.ds(c * CHUNK, HALF)]
        idxi_hi = iidall.at[pl.ds(c * CHUNK + HALF, HALF)]
        idxu = uidall.at[pl.ds(c * CHUNK, CHUNK)]
        idxi = iidall.at[pl.ds(c * CHUNK, CHUNK)]
        return (
            pltpu.async_copy(uf.at[idxu_lo], urows.at[b, pl.ds(0, HALF)],
                             sem_ul[b]),
            pltpu.async_copy(vf.at[idxi_lo], vrows.at[b, pl.ds(0, HALF)],
                             sem_vl[b]),
            pltpu.async_copy(ub.at[idxu], ubias.at[b], sem_ub[b]),
            pltpu.async_copy(ib.at[idxi], ibias.at[b], sem_ib[b]),
            pltpu.async_copy(uf.at[idxu_hi], urows.at[b, pl.ds(HALF, HALF)],
                             sem_uh[b]),
            pltpu.async_copy(vf.at[idxi_hi], vrows.at[b, pl.ds(HALF, HALF)],
                             sem_vh[b]),
        )

    cuid0.wait()
    ciid0.wait()
    descs = [None, None]
    descs[0] = issue(0)
    cuidr.wait()
    ciidr.wait()
    cg.wait()
    lane_iota = lax.iota(jnp.int32, LANES)
    gbvec = plsc.load_gather(gbv, [jnp.zeros((LANES,), jnp.int32)])

    for c in range(NCHUNK):
        b = c % 2
        if c + 1 < NCHUNK:
            descs[1 - b] = issue(c + 1)
        (du_lo, dv_lo, dub, dib, du_hi, dv_hi) = descs[b]

        def make_group_body(b, c):
            def group_body(g, inner):
                svec = jnp.zeros((LANES,), jnp.float32)
                for r in range(LANES):
                    row = g * LANES + r
                    acc = (urows[b, row, pl.ds(0, LANES)]
                           * vrows[b, row, pl.ds(0, LANES)])
                    for j in range(1, EMB // LANES):
                        acc = acc + (urows[b, row, pl.ds(j * LANES, LANES)]
                                     * vrows[b, row, pl.ds(j * LANES, LANES)])
                    svec = jnp.where(lane_iota == r, jnp.sum(acc), svec)
                p = (svec + ubias[b, pl.ds(g * LANES, LANES)]
                     + ibias[b, pl.ds(g * LANES, LANES)] + gbvec)
                off = c * CHUNK + g * LANES
                predall[pl.ds(off, LANES)] = p
                ctrall[pl.ds(off, LANES)] = 1.0 / (1.0 + jnp.exp(-p))
                return inner
            return group_body

        body = make_group_body(b, c)
        du_lo.wait()
        dv_lo.wait()
        dub.wait()
        dib.wait()
        lax.fori_loop(0, HGROUPS, body, 0)
        du_hi.wait()
        dv_hi.wait()
        lax.fori_loop(HGROUPS, GROUPS, body, 0)

    wp = pltpu.async_copy(predall, pred_out.at[pl.ds(base, PER_W)], sem_wp)
    wc = pltpu.async_copy(ctrall, ctr_out.at[pl.ds(base, PER_W)], sem_wc)
    wp.wait()
    wc.wait()


@jax.jit
def _fm_call(uid, iid, uf, vf, ub, ib, gb):
    mesh = plsc.VectorSubcoreMesh(core_axis_name="c", subcore_axis_name="s")
    f32 = jnp.float32
    run = functools.partial(
        pl.kernel,
        mesh=mesh,
        compiler_params=pltpu.CompilerParams(
            needs_layout_passes=False,
            skip_device_barrier=True,
            disable_bounds_checks=True,
            disable_semaphore_checks=True,
        ),
        out_type=[
            jax.ShapeDtypeStruct((B,), f32),
            jax.ShapeDtypeStruct((B,), f32),
        ],
        scratch_types=[
            pltpu.VMEM((PER_W,), jnp.int32),
            pltpu.VMEM((PER_W,), jnp.int32),
            pltpu.VMEM((2, CHUNK, EMB), f32),
            pltpu.VMEM((2, CHUNK, EMB), f32),
            pltpu.VMEM((2, CHUNK), f32),
            pltpu.VMEM((2, CHUNK), f32),
            pltpu.VMEM((PER_W,), f32),
            pltpu.VMEM((PER_W,), f32),
            pltpu.VMEM((1,), f32),
        ] + [pltpu.SemaphoreType.DMA] * 19,
    )(_fm_body)
    return run(uf, vf, ub, ib, uid, iid, gb)


def kernel(user_id, item_id, user_factors, item_factors, user_bias,
           item_bias, global_bias):
    uid = user_id.astype(jnp.int32)
    iid = item_id.astype(jnp.int32)
    pred, ctr = _fm_call(uid, iid, user_factors, item_factors,
                         user_bias, item_bias,
                         global_bias.astype(jnp.float32))
    return (pred, ctr)


# speedup vs baseline: 1.0491x; 1.0491x over previous
import functools

import jax
import jax.numpy as jnp
from jax import lax
from jax.experimental import pallas as pl
from jax.experimental.pallas import tpu as pltpu
from jax.experimental.pallas import tpu_sc as plsc

B = 16384
EMB = 128
NC = 2
NS = 16
NW = NC * NS
PER_W = B // NW
CHUNK = 128
NCHUNK = PER_W // CHUNK
LANES = 16
GROUPS = CHUNK // LANES


def _fm_body(uf, vf, ub, ib, uid, iid, gb,
             pred_out, ctr_out,
             uidall, iidall, urows, vrows, ubias, ibias, predall,
             ctrall, gbv,
             sem_u0, sem_u1, sem_v0, sem_v1,
             sem_ub0, sem_ub1, sem_ib0, sem_ib1,
             sem_wp, sem_wc, sem_p0, sem_p1, sem_p2):
    cid = lax.axis_index("c")
    sid = lax.axis_index("s")
    wid = sid * NC + cid
    base = wid * PER_W

    sem_u = (sem_u0, sem_u1)
    sem_v = (sem_v0, sem_v1)
    sem_ub = (sem_ub0, sem_ub1)
    sem_ib = (sem_ib0, sem_ib1)

    cg = pltpu.async_copy(gb, gbv, sem_p0)
    cuid = pltpu.async_copy(uid.at[pl.ds(base, PER_W)], uidall, sem_p1)
    ciid = pltpu.async_copy(iid.at[pl.ds(base, PER_W)], iidall, sem_p2)
    cuid.wait()
    ciid.wait()
    cg.wait()
    lane_iota = lax.iota(jnp.int32, LANES)
    gbvec = plsc.load_gather(gbv, [jnp.zeros((LANES,), jnp.int32)])

    def issue(c):
        b = c % 2
        idxu = uidall.at[pl.ds(c * CHUNK, CHUNK)]
        idxi = iidall.at[pl.ds(c * CHUNK, CHUNK)]
        return (
            pltpu.async_copy(uf.at[idxu], urows.at[b], sem_u[b]),
            pltpu.async_copy(vf.at[idxi], vrows.at[b], sem_v[b]),
            pltpu.async_copy(ub.at[idxu], ubias.at[b], sem_ub[b]),
            pltpu.async_copy(ib.at[idxi], ibias.at[b], sem_ib[b]),
        )

    descs = [None, None]
    descs[0] = issue(0)

    for c in range(NCHUNK):
        b = c % 2
        if c + 1 < NCHUNK:
            descs[1 - b] = issue(c + 1)
        for d in descs[b]:
            d.wait()

        def group_body(g, inner, b=b, c=c):
            svec = jnp.zeros((LANES,), jnp.float32)
            for r in range(LANES):
                row = g * LANES + r
                acc = (urows[b, row, pl.ds(0, LANES)]
                       * vrows[b, row, pl.ds(0, LANES)])
                for j in range(1, EMB // LANES):
                    acc = acc + (urows[b, row, pl.ds(j * LANES, LANES)]
                                 * vrows[b, row, pl.ds(j * LANES, LANES)])
                svec = jnp.where(lane_iota == r, jnp.sum(acc), svec)
            p = (svec + ubias[b, pl.ds(g * LANES, LANES)]
                 + ibias[b, pl.ds(g * LANES, LANES)] + gbvec)
            off = c * CHUNK + g * LANES
            predall[pl.ds(off, LANES)] = p
            ctrall[pl.ds(off, LANES)] = 1.0 / (1.0 + jnp.exp(-p))
            return inner

        lax.fori_loop(0, GROUPS, group_body, 0)

    wp = pltpu.async_copy(predall, pred_out.at[pl.ds(base, PER_W)], sem_wp)
    wc = pltpu.async_copy(ctrall, ctr_out.at[pl.ds(base, PER_W)], sem_wc)
    wp.wait()
    wc.wait()


@jax.jit
def _fm_call(uid, iid, uf, vf, ub, ib, gb):
    mesh = plsc.VectorSubcoreMesh(core_axis_name="c", subcore_axis_name="s")
    f32 = jnp.float32
    run = functools.partial(
        pl.kernel,
        mesh=mesh,
        compiler_params=pltpu.CompilerParams(
            needs_layout_passes=False,
            skip_device_barrier=True,
            disable_bounds_checks=True,
            disable_semaphore_checks=True,
            vmem_limit_bytes=131072,
        ),
        out_type=[
            jax.ShapeDtypeStruct((B,), f32),
            jax.ShapeDtypeStruct((B,), f32),
        ],
        scratch_types=[
            pltpu.VMEM((PER_W,), jnp.int32),
            pltpu.VMEM((PER_W,), jnp.int32),
            pltpu.VMEM((2, CHUNK, EMB), f32),
            pltpu.VMEM((2, CHUNK, EMB), f32),
            pltpu.VMEM((2, CHUNK), f32),
            pltpu.VMEM((2, CHUNK), f32),
            pltpu.VMEM((PER_W,), f32),
            pltpu.VMEM((PER_W,), f32),
            pltpu.VMEM((1,), f32),
        ] + [pltpu.SemaphoreType.DMA] * 13,
    )(_fm_body)
    return run(uf, vf, ub, ib, uid, iid, gb)


def kernel(user_id, item_id, user_factors, item_factors, user_bias,
           item_bias, global_bias):
    uid = user_id.astype(jnp.int32)
    iid = item_id.astype(jnp.int32)
    pred, ctr = _fm_call(uid, iid, user_factors, item_factors,
                         user_bias, item_bias,
                         global_bias.astype(jnp.float32))
    return (pred, ctr)
